# Initial kernel scaffold; baseline (speedup 1.0000x reference)
import jax
import jax.numpy as jnp
from jax.experimental import pallas as pl


def kernel(x):
    raise NotImplementedError("write your pallas kernel here")


# speedup vs baseline: 64.7968x; 64.7968x over previous
import jax
import jax.numpy as jnp
from jax.experimental import pallas as pl
from jax.experimental.pallas import tpu as pltpu

_C = 384
_H = 224
_HF = 112
_CBLK = 16


def _shift_body(x_ref, o_ref):
    c = pl.program_id(1)
    g = jnp.clip((c * _CBLK - 192) // 48, 0, 3)

    @pl.when(g == 0)
    def _():
        o_ref[...] = x_ref[...]

    @pl.when(g == 1)
    def _():
        o_ref[:, :, :_HF, :_HF] = x_ref[:, :, :_HF, _HF:]
        o_ref[:, :, :_HF, _HF:] = x_ref[:, :, _HF:, :_HF]
        o_ref[:, :, _HF:, :_HF] = x_ref[:, :, _HF:, _HF:]
        o_ref[:, :, _HF:, _HF:] = x_ref[:, :, :_HF, :_HF]

    @pl.when(g == 2)
    def _():
        o_ref[:, :, :_HF, :] = x_ref[:, :, _HF:, :]
        o_ref[:, :, _HF:, :] = x_ref[:, :, :_HF, :]

    @pl.when(g == 3)
    def _():
        o_ref[:, :, :_HF, :_HF] = x_ref[:, :, _HF:, _HF:]
        o_ref[:, :, :_HF, _HF:] = x_ref[:, :, :_HF, :_HF]
        o_ref[:, :, _HF:, :_HF] = x_ref[:, :, :_HF, _HF:]
        o_ref[:, :, _HF:, _HF:] = x_ref[:, :, _HF:, :_HF]


def kernel(x):
    b, c, h, w = x.shape
    grid = (b, c // _CBLK)
    spec = pl.BlockSpec((1, _CBLK, h, w), lambda i, j: (i, j, 0, 0))
    return pl.pallas_call(
        _shift_body,
        grid=grid,
        in_specs=[spec],
        out_specs=spec,
        out_shape=jax.ShapeDtypeStruct(x.shape, x.dtype),
        compiler_params=pltpu.CompilerParams(
            dimension_semantics=("parallel", "parallel"),
        ),
    )(x)
